# baseline (speedup 1.0000x reference)
import functools

import jax
import jax.numpy as jnp
from jax import lax
from jax.experimental import pallas as pl
from jax.experimental.pallas import tpu as pltpu
from jax.experimental.pallas import tpu_sc as plsc

NC = 2
NS = 16
NW = NC * NS
NB = 8


@functools.partial(jax.jit, static_argnums=())
def _gather_rows(indices, table):
    mesh = plsc.VectorSubcoreMesh(core_axis_name="c", subcore_axis_name="s")
    b, h = indices.shape
    v, d = table.shape
    b_per_w = b // NW
    nsteps = b_per_w // NB

    @functools.partial(
        pl.kernel,
        out_type=jax.ShapeDtypeStruct((b, h, d), jnp.float32),
        mesh=mesh,
        scratch_types=[
            pltpu.VMEM((b_per_w, h), jnp.int32),
            pltpu.VMEM((NB, h, d), jnp.float32),
            pltpu.VMEM((NB, h, d), jnp.float32),
            pltpu.SemaphoreType.DMA,
            pltpu.SemaphoreType.DMA,
            pltpu.SemaphoreType.DMA,
            pltpu.SemaphoreType.DMA,
        ],
        compiler_params=pltpu.CompilerParams(use_tc_tiling_on_sc=False),
    )
    def run(tab_hbm, idx_hbm, out_hbm, idx_v, buf0, buf1, g0, g1, o0, o1):
        wid = lax.axis_index("s") * NC + lax.axis_index("c")
        b0 = wid * b_per_w
        pltpu.sync_copy(idx_hbm.at[pl.ds(b0, b_per_w)], idx_v)

        bufs = (buf0, buf1)
        gsems = (g0, g1)
        osems = (o0, o1)

        def fire_g(s, bsel):
            for t in range(NB):
                pltpu.async_copy(
                    tab_hbm.at[idx_v.at[s * NB + t]],
                    bufs[bsel].at[t],
                    gsems[bsel],
                )

        def drain_g(bsel):
            pltpu.make_async_copy(
                out_hbm.at[pl.ds(0, NB)], bufs[bsel], gsems[bsel]
            ).wait()

        def fire_o(s, bsel):
            pltpu.async_copy(
                bufs[bsel], out_hbm.at[pl.ds(b0 + s * NB, NB)], osems[bsel]
            )

        def wait_o(bsel):
            pltpu.make_async_copy(
                bufs[bsel], out_hbm.at[pl.ds(0, NB)], osems[bsel]
            ).wait()

        fire_g(0, 0)
        fire_g(1, 1)
        drain_g(0)
        fire_o(0, 0)

        def body(i, carry):
            drain_g(1)
            fire_o(2 * i + 1, 1)
            wait_o(0)
            fire_g(2 * i + 2, 0)
            drain_g(0)
            fire_o(2 * i + 2, 0)
            wait_o(1)
            fire_g(2 * i + 3, 1)
            return carry

        lax.fori_loop(0, (nsteps - 2) // 2, body, 0)

        drain_g(1)
        wait_o(0)
        fire_o(nsteps - 1, 1)
        wait_o(1)

    return run(table, indices)


def kernel(indices, table):
    b, h = indices.shape
    assert b % (NW * NB) == 0 and (b // NW // NB) % 2 == 0
    return _gather_rows(indices.astype(jnp.int32), table)
